# baseline (speedup 1.0000x reference)
import functools

import jax
import jax.numpy as jnp
from jax import lax
from jax.experimental import pallas as pl
from jax.experimental.pallas import tpu as pltpu
from jax.experimental.pallas import tpu_sc as plsc

DIM = 128
NUM_CORES = 2
NUM_SUBCORES = 16
NUM_WORKERS = NUM_CORES * NUM_SUBCORES
CHUNK = 200
NBUF = 4
DEFER = 2
PEEL_TAIL = NBUF


def kernel(input, table):
    batch, seq = input.shape
    num_idx = batch * seq
    idx = input.astype(jnp.int32).T.reshape(num_idx)

    b_per_w = num_idx // NUM_WORKERS
    n_chunks = b_per_w // CHUNK
    assert b_per_w * NUM_WORKERS == num_idx
    assert n_chunks * CHUNK == b_per_w and n_chunks % NBUF == 0
    assert n_chunks >= NBUF + PEEL_TAIL and NBUF > DEFER

    mesh = plsc.VectorSubcoreMesh(core_axis_name="c", subcore_axis_name="s")

    @functools.partial(
        pl.kernel,
        mesh=mesh,
        out_type=jax.ShapeDtypeStruct((num_idx, DIM), jnp.float32),
        scratch_types=(
            [pltpu.VMEM((CHUNK,), jnp.int32) for _ in range(NBUF)]
            + [pltpu.VMEM((CHUNK, DIM), jnp.float32) for _ in range(NBUF)]
            + [pltpu.SemaphoreType.DMA for _ in range(3 * NBUF)]
        ),
    )
    def gather_kernel(table_hbm, idx_hbm, out_hbm, *scratch):
        idx_v = scratch[:NBUF]
        rows_v = scratch[NBUF:2 * NBUF]
        g_sem = scratch[2 * NBUF:3 * NBUF]
        o_sem = scratch[3 * NBUF:4 * NBUF]
        i_sem = scratch[4 * NBUF:]
        wid = lax.axis_index("s") * NUM_CORES + lax.axis_index("c")
        base = wid * b_per_w

        def idx_copy(off, b):
            return pltpu.make_async_copy(idx_hbm.at[pl.ds(off, CHUNK)],
                                         idx_v[b], i_sem[b])

        def gather_copy(b):
            return pltpu.make_async_copy(table_hbm.at[idx_v[b]], rows_v[b],
                                         g_sem[b])

        def out_copy(off, b):
            return pltpu.make_async_copy(rows_v[b],
                                         out_hbm.at[pl.ds(off, CHUNK)],
                                         o_sem[b])

        def step(c, b, prefetch, drain_fire):
            gather_copy(b).wait()
            if prefetch:
                idx_copy(base + (c + NBUF) * CHUNK, b).start()
            out_copy(base + c * CHUNK, b).start()
            if drain_fire:
                cd = c - DEFER
                bd = (b - DEFER) % NBUF
                out_copy(base + cd * CHUNK, bd).wait()
                idx_copy(base + (cd + NBUF) * CHUNK, bd).wait()
                gather_copy(bd).start()

        for b in range(NBUF):
            idx_copy(base + b * CHUNK, b).start()
            idx_copy(base + b * CHUNK, b).wait()
            gather_copy(b).start()

        for c in range(NBUF):
            step(c, c % NBUF, prefetch=True, drain_fire=c >= DEFER)

        @pl.loop(NBUF, n_chunks - PEEL_TAIL, step=NBUF)
        def _(j):
            for b in range(NBUF):
                step(j + b, b, prefetch=True, drain_fire=True)

        for c in range(n_chunks - PEEL_TAIL, n_chunks):
            step(c, c % NBUF,
                 prefetch=c + NBUF < n_chunks,
                 drain_fire=c - DEFER + NBUF < n_chunks)

        for c in range(n_chunks - NBUF, n_chunks):
            out_copy(base + c * CHUNK, c % NBUF).wait()

    out = gather_kernel(table, idx)
    return out.reshape(seq, batch, DIM).transpose(1, 0, 2)


# speedup vs baseline: 1.0141x; 1.0141x over previous
import functools

import jax
import jax.numpy as jnp
from jax import lax
from jax.experimental import pallas as pl
from jax.experimental.pallas import tpu as pltpu
from jax.experimental.pallas import tpu_sc as plsc

DIM = 128
NUM_CORES = 2
NUM_SUBCORES = 16
NUM_WORKERS = NUM_CORES * NUM_SUBCORES
CHUNK = 200
NBUF = 4


def kernel(input, table):
    batch, seq = input.shape
    num_idx = batch * seq
    idx = input.astype(jnp.int32).T.reshape(num_idx)

    b_per_w = num_idx // NUM_WORKERS
    n_chunks = b_per_w // CHUNK
    assert b_per_w * NUM_WORKERS == num_idx
    assert n_chunks * CHUNK == b_per_w and n_chunks % NBUF == 0

    mesh = plsc.VectorSubcoreMesh(core_axis_name="c", subcore_axis_name="s")

    @functools.partial(
        pl.kernel,
        mesh=mesh,
        out_type=jax.ShapeDtypeStruct((num_idx, DIM), jnp.float32),
        scratch_types=(
            [pltpu.VMEM((CHUNK,), jnp.int32) for _ in range(NBUF)]
            + [pltpu.VMEM((CHUNK, DIM), jnp.float32) for _ in range(NBUF)]
            + [pltpu.SemaphoreType.DMA for _ in range(3 * NBUF)]
        ),
    )
    def gather_kernel(table_hbm, idx_hbm, out_hbm, *scratch):
        idx_v = scratch[:NBUF]
        rows_v = scratch[NBUF:2 * NBUF]
        g_sem = scratch[2 * NBUF:3 * NBUF]
        o_sem = scratch[3 * NBUF:4 * NBUF]
        i_sem = scratch[4 * NBUF:]
        wid = lax.axis_index("s") * NUM_CORES + lax.axis_index("c")
        base = wid * b_per_w

        def idx_copy(off, b):
            return pltpu.make_async_copy(idx_hbm.at[pl.ds(off, CHUNK)],
                                         idx_v[b], i_sem[b])

        def gather_copy(b):
            return pltpu.make_async_copy(table_hbm.at[idx_v[b]], rows_v[b],
                                         g_sem[b])

        def out_copy(off, b):
            return pltpu.make_async_copy(rows_v[b],
                                         out_hbm.at[pl.ds(off, CHUNK)],
                                         o_sem[b])

        def step(off, b, issue_next):
            gather_copy(b).wait()
            if issue_next:
                idx_copy(off + NBUF * CHUNK, b).start()
            out_copy(off, b).start()
            if issue_next:
                out_copy(off, b).wait()
                idx_copy(off + NBUF * CHUNK, b).wait()
                gather_copy(b).start()

        for b in range(NBUF):
            idx_copy(base + b * CHUNK, b).start()
            idx_copy(base + b * CHUNK, b).wait()
            gather_copy(b).start()

        @pl.loop(0, n_chunks - NBUF, step=NBUF)
        def _(j):
            for b in range(NBUF):
                step(base + (j + b) * CHUNK, b, issue_next=True)

        for b in range(NBUF):
            off = base + (n_chunks - NBUF + b) * CHUNK
            step(off, b, issue_next=False)
            out_copy(off, b).wait()

    out = gather_kernel(table, idx)
    return out.reshape(seq, batch, DIM).transpose(1, 0, 2)
